# baseline (speedup 1.0000x reference)
import functools

import jax
import jax.numpy as jnp
from jax import lax
from jax.experimental import pallas as pl
from jax.experimental.pallas import tpu as pltpu
from jax.experimental.pallas import tpu_sc as plsc

NE = 8
NTOK = 2048
DM = 768
DF = 3072
BM = 128
NBLK = (2 * NTOK) // BM + NE
NROWS = NBLK * BM

NC = 2
NS = 16
NW = NC * NS
TPW = NTOK // NW
LANES = 16


def _router_body(x_ref, wg_ref, bg_ref, s0_ref, s1_ref, w0_ref, w1_ref,
                 bexp_ref):
    xf = x_ref[...]
    logits = jnp.dot(xf, wg_ref[...], preferred_element_type=jnp.float32)
    logits = logits + bg_ref[...]

    e_ids = lax.broadcasted_iota(jnp.int32, (NTOK, NE), 1)
    m1 = jnp.max(logits, axis=1, keepdims=True)
    i1 = jnp.argmax(logits, axis=1)
    oh1 = (e_ids == i1[:, None]).astype(jnp.float32)
    masked = jnp.where(oh1 > 0.0, jnp.float32(-1e30), logits)
    m2 = jnp.max(masked, axis=1, keepdims=True)
    i2 = jnp.argmax(masked, axis=1)
    oh2 = (e_ids == i2[:, None]).astype(jnp.float32)

    d = jnp.exp(m2[:, 0] - m1[:, 0])
    w1v = 1.0 / (1.0 + d)
    w2v = 1.0 - w1v

    PCH, CCH = NTOK // 16, 16
    p_i = lax.broadcasted_iota(jnp.int32, (PCH, PCH), 0)
    p_j = lax.broadcasted_iota(jnp.int32, (PCH, PCH), 1)
    ltrip = (p_j < p_i).astype(jnp.float32)

    def _ranks(oh):
        x3 = oh.reshape(PCH, CCH, NE)
        s = jnp.dot(ltrip, jnp.sum(x3, axis=1),
                    preferred_element_type=jnp.float32)
        acc = x3
        for sh in (1, 2, 4, 8):
            acc = acc + jnp.concatenate(
                [jnp.zeros((PCH, sh, NE), jnp.float32), acc[:, :-sh, :]],
                axis=1)
        t_excl = acc - x3
        return (s[:, None, :] + t_excl).reshape(NTOK, NE)

    tot0 = jnp.sum(oh1, axis=0)
    rank0 = jnp.sum(_ranks(oh1) * oh1, axis=1)
    rank1 = jnp.sum((_ranks(oh2) + tot0[None, :]) * oh2, axis=1)

    counts = tot0 + jnp.sum(oh2, axis=0)
    nblocks = jnp.ceil(counts * (1.0 / BM))
    a_i = lax.broadcasted_iota(jnp.int32, (NE, NE), 0)
    a_j = lax.broadcasted_iota(jnp.int32, (NE, NE), 1)
    ltri8 = (a_j < a_i).astype(jnp.float32)
    bstart = jnp.dot(ltri8, nblocks[:, None],
                     preferred_element_type=jnp.float32)[:, 0]
    bend = bstart + nblocks

    base0 = jnp.dot(oh1, bstart[:, None],
                    preferred_element_type=jnp.float32)[:, 0] * BM
    base1 = jnp.dot(oh2, bstart[:, None],
                    preferred_element_type=jnp.float32)[:, 0] * BM
    s0_ref[...] = (base0 + rank0).astype(jnp.int32)[None, :]
    s1_ref[...] = (base1 + rank1).astype(jnp.int32)[None, :]
    w0_ref[...] = jnp.broadcast_to(w1v[:, None], (NTOK, LANES))
    w1_ref[...] = jnp.broadcast_to(w2v[:, None], (NTOK, LANES))

    t_ids = lax.broadcasted_iota(jnp.int32, (NBLK, NE), 0).astype(jnp.float32)
    bexp = jnp.sum((t_ids >= bend[None, :]).astype(jnp.int32), axis=1)
    bexp = jnp.minimum(bexp, NE - 1)
    used = jnp.sum(nblocks).astype(jnp.int32)
    e_iota = lax.broadcasted_iota(jnp.int32, (NE,), 0)
    laste = jnp.max(jnp.where(counts > 0.0, e_iota, -1))
    t_blk = lax.broadcasted_iota(jnp.int32, (NBLK,), 0)
    bexp = jnp.where(t_blk < used, bexp, laste)

    present = (counts > 0.0).astype(jnp.float32)
    run_ord = jnp.dot(ltri8, present[:, None],
                      preferred_element_type=jnp.float32)[:, 0]
    par_e = run_ord - 2.0 * jnp.floor(run_ord * 0.5)
    e_f = e_iota.astype(jnp.float32)
    cand = jnp.where((present[None, :] > 0.0)
                     & (e_f[None, :] > e_f[:, None]),
                     e_f[None, :], jnp.float32(NE + 1))
    nxt_e = jnp.min(cand, axis=1)
    nxt_e = jnp.where(nxt_e > NE, -1.0, nxt_e)
    blk_oh = (bexp[:, None] ==
              lax.broadcasted_iota(jnp.int32, (NBLK, NE), 1)
              ).astype(jnp.float32)
    par_t = jnp.dot(blk_oh, par_e[:, None],
                    preferred_element_type=jnp.float32)[:, 0].astype(jnp.int32)
    nxt_t = jnp.dot(blk_oh, nxt_e[:, None],
                    preferred_element_type=jnp.float32)[:, 0].astype(jnp.int32)
    prev = jnp.concatenate([jnp.full((1,), -1, jnp.int32), bexp[:-1]])
    first_t = ((bexp != prev) & (t_blk < used)).astype(jnp.int32)
    bexp_ref[...] = jnp.concatenate(
        [bexp, used[None], first_t, par_t, nxt_t])[None, :]


def _router(xf, Wg, bg2):
    return pl.pallas_call(
        _router_body,
        out_shape=(
            jax.ShapeDtypeStruct((1, NTOK), jnp.int32),
            jax.ShapeDtypeStruct((1, NTOK), jnp.int32),
            jax.ShapeDtypeStruct((NTOK, LANES), jnp.float32),
            jax.ShapeDtypeStruct((NTOK, LANES), jnp.float32),
            jax.ShapeDtypeStruct((1, 4 * NBLK + 1), jnp.int32),
        ),
    )(xf, Wg, bg2)


_SC_MESH = plsc.VectorSubcoreMesh(core_axis_name="c", subcore_axis_name="s")


@functools.partial(
    pl.kernel,
    out_type=jax.ShapeDtypeStruct((NROWS, DM), jnp.float32),
    mesh=_SC_MESH,
    scratch_types=[
        [pltpu.VMEM((TPW // 2,), jnp.int32) for _ in range(4)],
        [pltpu.VMEM((TPW // 2, DM), jnp.float32) for _ in range(2)],
        [pltpu.SemaphoreType.DMA for _ in range(2)],
    ],
)
def _dispatch(x_hbm, s0_hbm, s1_hbm, xs_hbm, s_v, rows_v, sems):
    wid = lax.axis_index("s") * NC + lax.axis_index("c")
    base = wid * TPW
    half = TPW // 2
    cps = []
    for hb in range(2):
        hbase = base + hb * half
        pltpu.sync_copy(s0_hbm.at[pl.ds(hbase, half)], s_v[2 * hb])
        pltpu.sync_copy(s1_hbm.at[pl.ds(hbase, half)], s_v[2 * hb + 1])
        pltpu.sync_copy(x_hbm.at[pl.ds(hbase, half)], rows_v[hb])
        cps.append(pltpu.async_copy(rows_v[hb], xs_hbm.at[s_v[2 * hb]],
                                    sems[hb]))
        cps.append(pltpu.async_copy(rows_v[hb], xs_hbm.at[s_v[2 * hb + 1]],
                                    sems[hb]))
    for cp in cps:
        cp.wait()


def _ffn_body(meta_ref, x_ref, w1_hbm, b1_ref, w2_hbm, b2_ref, o_ref,
              w1a, w2a, w1b, w2b, sema1, sema2, semb1, semb2):
    t = pl.program_id(0)
    used = meta_ref[NBLK]
    e = meta_ref[t]
    first = meta_ref[NBLK + 1 + t]
    par = meta_ref[2 * NBLK + 1 + t]
    nxt = meta_ref[3 * NBLK + 1 + t]

    @pl.when(t == 0)
    def _():
        pltpu.make_async_copy(w1_hbm.at[e], w1a, sema1).start()
        pltpu.make_async_copy(w2_hbm.at[e], w2a, sema2).start()

    @pl.when(first == 1)
    def _():
        @pl.when((par == 0) & (nxt >= 0))
        def _():
            pltpu.make_async_copy(w1_hbm.at[nxt], w1b, semb1).start()
            pltpu.make_async_copy(w2_hbm.at[nxt], w2b, semb2).start()

        @pl.when((par == 1) & (nxt >= 0))
        def _():
            pltpu.make_async_copy(w1_hbm.at[nxt], w1a, sema1).start()
            pltpu.make_async_copy(w2_hbm.at[nxt], w2a, sema2).start()

    @pl.when(t < used)
    def _():
        xb = x_ref[...]

        @pl.when(par == 0)
        def _():
            @pl.when(first == 1)
            def _():
                pltpu.make_async_copy(w1_hbm.at[e], w1a, sema1).wait()

            h = jnp.dot(xb, w1a[...], preferred_element_type=jnp.float32)
            h = h + b1_ref[0]
            h = 0.5 * h * (
                1.0 + lax.erf(h * jnp.float32(0.7071067811865476)))

            @pl.when(first == 1)
            def _():
                pltpu.make_async_copy(w2_hbm.at[e], w2a, sema2).wait()

            y = jnp.dot(h, w2a[...], preferred_element_type=jnp.float32)
            o_ref[...] = y + b2_ref[0]

        @pl.when(par == 1)
        def _():
            @pl.when(first == 1)
            def _():
                pltpu.make_async_copy(w1_hbm.at[e], w1b, semb1).wait()

            h = jnp.dot(xb, w1b[...], preferred_element_type=jnp.float32)
            h = h + b1_ref[0]
            h = 0.5 * h * (
                1.0 + lax.erf(h * jnp.float32(0.7071067811865476)))

            @pl.when(first == 1)
            def _():
                pltpu.make_async_copy(w2_hbm.at[e], w2b, semb2).wait()

            y = jnp.dot(h, w2b[...], preferred_element_type=jnp.float32)
            o_ref[...] = y + b2_ref[0]


def _ffn(bexp, xs, W1, b1, W2, b2):
    grid_spec = pltpu.PrefetchScalarGridSpec(
        num_scalar_prefetch=1,
        grid=(NBLK,),
        in_specs=[
            pl.BlockSpec((BM, DM),
                         lambda t, be: (jnp.minimum(t, be[NBLK] - 1), 0)),
            pl.BlockSpec(memory_space=pl.ANY),
            pl.BlockSpec((1, 1, DF), lambda t, be: (be[t], 0, 0)),
            pl.BlockSpec(memory_space=pl.ANY),
            pl.BlockSpec((1, 1, DM), lambda t, be: (be[t], 0, 0)),
        ],
        out_specs=pl.BlockSpec(
            (BM, DM), lambda t, be: (jnp.minimum(t, be[NBLK] - 1), 0)),
        scratch_shapes=[
            pltpu.VMEM((DM, DF), jnp.float32),
            pltpu.VMEM((DF, DM), jnp.float32),
            pltpu.VMEM((DM, DF), jnp.float32),
            pltpu.VMEM((DF, DM), jnp.float32),
            pltpu.SemaphoreType.DMA,
            pltpu.SemaphoreType.DMA,
            pltpu.SemaphoreType.DMA,
            pltpu.SemaphoreType.DMA,
        ],
    )
    return pl.pallas_call(
        _ffn_body,
        grid_spec=grid_spec,
        out_shape=jax.ShapeDtypeStruct((NROWS, DM), jnp.float32),
    )(bexp, xs, W1, b1.reshape(NE, 1, DF), W2, b2.reshape(NE, 1, DM))


HPW = TPW // 2


@functools.partial(
    pl.kernel,
    out_type=jax.ShapeDtypeStruct((NTOK, DM), jnp.float32),
    mesh=_SC_MESH,
    scratch_types=[
        [pltpu.VMEM((HPW,), jnp.int32) for _ in range(4)],
        [pltpu.VMEM((HPW, LANES), jnp.float32) for _ in range(4)],
        [pltpu.VMEM((HPW, DM), jnp.float32) for _ in range(4)],
        [pltpu.SemaphoreType.DMA for _ in range(2)],
    ],
)
def _combine(y_hbm, s0_hbm, s1_hbm, w0_hbm, w1_hbm, o_hbm,
             s_v, w_v, y_v, sems):
    wid = lax.axis_index("s") * NC + lax.axis_index("c")
    base = wid * TPW
    cps = []
    for hb in range(2):
        hbase = base + hb * HPW
        pltpu.sync_copy(s0_hbm.at[pl.ds(hbase, HPW)], s_v[2 * hb])
        pltpu.sync_copy(s1_hbm.at[pl.ds(hbase, HPW)], s_v[2 * hb + 1])
        pltpu.sync_copy(w0_hbm.at[pl.ds(hbase, HPW)], w_v[2 * hb])
        pltpu.sync_copy(w1_hbm.at[pl.ds(hbase, HPW)], w_v[2 * hb + 1])
        cps.append(pltpu.async_copy(y_hbm.at[s_v[2 * hb]],
                                    y_v[2 * hb], sems[hb]))
        cps.append(pltpu.async_copy(y_hbm.at[s_v[2 * hb + 1]],
                                    y_v[2 * hb + 1], sems[hb]))

    for hb in range(2):
        cps[2 * hb].wait()
        cps[2 * hb + 1].wait()
        y0_v, y1_v = y_v[2 * hb], y_v[2 * hb + 1]
        w0_v, w1_v = w_v[2 * hb], w_v[2 * hb + 1]

        def body(j, carry):
            wj0 = w0_v[j, :]
            wj1 = w1_v[j, :]
            for c in range(DM // LANES):
                sl = pl.ds(c * LANES, LANES)
                y0_v[j, sl] = y0_v[j, sl] * wj0 + y1_v[j, sl] * wj1
            return carry

        lax.fori_loop(0, HPW, body, 0)
        pltpu.sync_copy(y0_v, o_hbm.at[pl.ds(base + hb * HPW, HPW)])


def kernel(x, Wg, bg, W1, b1, W2, b2):
    xf = x.reshape(NTOK, DM)
    s0, s1, w0, w1, bexp = _router(xf, Wg, bg.reshape(1, NE))
    s0 = s0.reshape(NTOK)
    s1 = s1.reshape(NTOK)
    xs = _dispatch(xf, s0, s1)
    y = _ffn(bexp.reshape(4 * NBLK + 1), xs, W1, b1, W2, b2)
    out = _combine(y, s0, s1, w0, w1)
    return out.reshape(x.shape)


# speedup vs baseline: 1.0690x; 1.0690x over previous
import functools

import jax
import jax.numpy as jnp
from jax import lax
from jax.experimental import pallas as pl
from jax.experimental.pallas import tpu as pltpu
from jax.experimental.pallas import tpu_sc as plsc

NE = 8
NTOK = 2048
DM = 768
DF = 3072
BM = 128
NBLK = (2 * NTOK) // BM + NE
NROWS = NBLK * BM

NC = 2
NS = 16
NW = NC * NS
TPW = NTOK // NW
LANES = 16


def _router_body(x_ref, wg_ref, bg_ref, s0_ref, s1_ref, w0_ref, w1_ref,
                 bexp_ref):
    xf = x_ref[...]
    logits = jnp.dot(xf, wg_ref[...], preferred_element_type=jnp.float32)
    logits = logits + bg_ref[...]

    e_ids = lax.broadcasted_iota(jnp.int32, (NTOK, NE), 1)
    m1 = jnp.max(logits, axis=1, keepdims=True)
    i1 = jnp.argmax(logits, axis=1)
    oh1 = (e_ids == i1[:, None]).astype(jnp.float32)
    masked = jnp.where(oh1 > 0.0, jnp.float32(-1e30), logits)
    m2 = jnp.max(masked, axis=1, keepdims=True)
    i2 = jnp.argmax(masked, axis=1)
    oh2 = (e_ids == i2[:, None]).astype(jnp.float32)

    d = jnp.exp(m2[:, 0] - m1[:, 0])
    w1v = 1.0 / (1.0 + d)
    w2v = 1.0 - w1v

    PCH, CCH = NTOK // 16, 16
    p_i = lax.broadcasted_iota(jnp.int32, (PCH, PCH), 0)
    p_j = lax.broadcasted_iota(jnp.int32, (PCH, PCH), 1)
    ltrip = (p_j < p_i).astype(jnp.float32)

    def _ranks(oh):
        x3 = oh.reshape(PCH, CCH, NE)
        s = jnp.dot(ltrip, jnp.sum(x3, axis=1),
                    preferred_element_type=jnp.float32)
        acc = x3
        for sh in (1, 2, 4, 8):
            acc = acc + jnp.concatenate(
                [jnp.zeros((PCH, sh, NE), jnp.float32), acc[:, :-sh, :]],
                axis=1)
        t_excl = acc - x3
        return (s[:, None, :] + t_excl).reshape(NTOK, NE)

    tot0 = jnp.sum(oh1, axis=0)
    rank0 = jnp.sum(_ranks(oh1) * oh1, axis=1)
    rank1 = jnp.sum((_ranks(oh2) + tot0[None, :]) * oh2, axis=1)

    counts = tot0 + jnp.sum(oh2, axis=0)
    nblocks = jnp.ceil(counts * (1.0 / BM))
    a_i = lax.broadcasted_iota(jnp.int32, (NE, NE), 0)
    a_j = lax.broadcasted_iota(jnp.int32, (NE, NE), 1)
    ltri8 = (a_j < a_i).astype(jnp.float32)
    bstart = jnp.dot(ltri8, nblocks[:, None],
                     preferred_element_type=jnp.float32)[:, 0]
    bend = bstart + nblocks

    base0 = jnp.dot(oh1, bstart[:, None],
                    preferred_element_type=jnp.float32)[:, 0] * BM
    base1 = jnp.dot(oh2, bstart[:, None],
                    preferred_element_type=jnp.float32)[:, 0] * BM
    s0_ref[...] = (base0 + rank0).astype(jnp.int32)[None, :]
    s1_ref[...] = (base1 + rank1).astype(jnp.int32)[None, :]
    w0_ref[...] = jnp.broadcast_to(w1v[:, None], (NTOK, LANES))
    w1_ref[...] = jnp.broadcast_to(w2v[:, None], (NTOK, LANES))

    t_ids = lax.broadcasted_iota(jnp.int32, (NBLK, NE), 0).astype(jnp.float32)
    bexp = jnp.sum((t_ids >= bend[None, :]).astype(jnp.int32), axis=1)
    bexp = jnp.minimum(bexp, NE - 1)
    used = jnp.sum(nblocks).astype(jnp.int32)
    e_iota = lax.broadcasted_iota(jnp.int32, (NE,), 0)
    laste = jnp.max(jnp.where(counts > 0.0, e_iota, -1))
    t_blk = lax.broadcasted_iota(jnp.int32, (NBLK,), 0)
    bexp = jnp.where(t_blk < used, bexp, laste)

    present = (counts > 0.0).astype(jnp.float32)
    run_ord = jnp.dot(ltri8, present[:, None],
                      preferred_element_type=jnp.float32)[:, 0]
    par_e = run_ord - 2.0 * jnp.floor(run_ord * 0.5)
    e_f = e_iota.astype(jnp.float32)
    cand = jnp.where((present[None, :] > 0.0)
                     & (e_f[None, :] > e_f[:, None]),
                     e_f[None, :], jnp.float32(NE + 1))
    nxt_e = jnp.min(cand, axis=1)
    nxt_e = jnp.where(nxt_e > NE, -1.0, nxt_e)
    blk_oh = (bexp[:, None] ==
              lax.broadcasted_iota(jnp.int32, (NBLK, NE), 1)
              ).astype(jnp.float32)
    par_t = jnp.dot(blk_oh, par_e[:, None],
                    preferred_element_type=jnp.float32)[:, 0].astype(jnp.int32)
    nxt_t = jnp.dot(blk_oh, nxt_e[:, None],
                    preferred_element_type=jnp.float32)[:, 0].astype(jnp.int32)
    prev = jnp.concatenate([jnp.full((1,), -1, jnp.int32), bexp[:-1]])
    first_t = ((bexp != prev) & (t_blk < used)).astype(jnp.int32)
    bexp_ref[...] = jnp.concatenate(
        [bexp, used[None], first_t, par_t, nxt_t])[None, :]


def _router(xf, Wg, bg2):
    return pl.pallas_call(
        _router_body,
        out_shape=(
            jax.ShapeDtypeStruct((1, NTOK), jnp.int32),
            jax.ShapeDtypeStruct((1, NTOK), jnp.int32),
            jax.ShapeDtypeStruct((NTOK, LANES), jnp.float32),
            jax.ShapeDtypeStruct((NTOK, LANES), jnp.float32),
            jax.ShapeDtypeStruct((1, 4 * NBLK + 1), jnp.int32),
        ),
    )(xf, Wg, bg2)


_SC_MESH = plsc.VectorSubcoreMesh(core_axis_name="c", subcore_axis_name="s")


@functools.partial(
    pl.kernel,
    out_type=jax.ShapeDtypeStruct((NROWS, DM), jnp.float32),
    mesh=_SC_MESH,
    scratch_types=[
        [pltpu.VMEM((TPW // 2,), jnp.int32) for _ in range(4)],
        [pltpu.VMEM((TPW // 2, DM), jnp.float32) for _ in range(2)],
        [pltpu.SemaphoreType.DMA for _ in range(2)],
    ],
)
def _dispatch(x_hbm, s0_hbm, s1_hbm, xs_hbm, s_v, rows_v, sems):
    wid = lax.axis_index("s") * NC + lax.axis_index("c")
    base = wid * TPW
    half = TPW // 2
    cps = []
    for hb in range(2):
        hbase = base + hb * half
        pltpu.sync_copy(s0_hbm.at[pl.ds(hbase, half)], s_v[2 * hb])
        pltpu.sync_copy(s1_hbm.at[pl.ds(hbase, half)], s_v[2 * hb + 1])
        pltpu.sync_copy(x_hbm.at[pl.ds(hbase, half)], rows_v[hb])
        cps.append(pltpu.async_copy(rows_v[hb], xs_hbm.at[s_v[2 * hb]],
                                    sems[hb]))
        cps.append(pltpu.async_copy(rows_v[hb], xs_hbm.at[s_v[2 * hb + 1]],
                                    sems[hb]))
    for cp in cps:
        cp.wait()


def _ffn_body(meta_ref, x_ref, w1_hbm, b1_ref, w2_hbm, b2_ref, o_ref,
              w1a, w2a, w1b, w2b, sema, semb):
    t = pl.program_id(0)
    used = meta_ref[NBLK]
    e = meta_ref[t]
    first = meta_ref[NBLK + 1 + t]
    par = meta_ref[2 * NBLK + 1 + t]
    nxt = meta_ref[3 * NBLK + 1 + t]

    @pl.when(t == 0)
    def _():
        pltpu.make_async_copy(w1_hbm.at[e], w1a, sema).start()
        pltpu.make_async_copy(w2_hbm.at[e], w2a, sema).start()

    @pl.when(first == 1)
    def _():
        @pl.when(par == 0)
        def _():
            pltpu.make_async_copy(w1_hbm.at[e], w1a, sema).wait()
            pltpu.make_async_copy(w2_hbm.at[e], w2a, sema).wait()

            @pl.when(nxt >= 0)
            def _():
                pltpu.make_async_copy(w1_hbm.at[nxt], w1b, semb).start()
                pltpu.make_async_copy(w2_hbm.at[nxt], w2b, semb).start()

        @pl.when(par == 1)
        def _():
            pltpu.make_async_copy(w1_hbm.at[e], w1b, semb).wait()
            pltpu.make_async_copy(w2_hbm.at[e], w2b, semb).wait()

            @pl.when(nxt >= 0)
            def _():
                pltpu.make_async_copy(w1_hbm.at[nxt], w1a, sema).start()
                pltpu.make_async_copy(w2_hbm.at[nxt], w2a, sema).start()

    @pl.when(t < used)
    def _():
        xb = x_ref[...]

        @pl.when(par == 0)
        def _():
            h = jnp.dot(xb, w1a[...], preferred_element_type=jnp.float32)
            h = h + b1_ref[0]
            h = 0.5 * h * (
                1.0 + lax.erf(h * jnp.float32(0.7071067811865476)))
            y = jnp.dot(h, w2a[...], preferred_element_type=jnp.float32)
            o_ref[...] = y + b2_ref[0]

        @pl.when(par == 1)
        def _():
            h = jnp.dot(xb, w1b[...], preferred_element_type=jnp.float32)
            h = h + b1_ref[0]
            h = 0.5 * h * (
                1.0 + lax.erf(h * jnp.float32(0.7071067811865476)))
            y = jnp.dot(h, w2b[...], preferred_element_type=jnp.float32)
            o_ref[...] = y + b2_ref[0]


def _ffn(bexp, xs, W1, b1, W2, b2):
    grid_spec = pltpu.PrefetchScalarGridSpec(
        num_scalar_prefetch=1,
        grid=(NBLK,),
        in_specs=[
            pl.BlockSpec((BM, DM),
                         lambda t, be: (jnp.minimum(t, be[NBLK] - 1), 0)),
            pl.BlockSpec(memory_space=pl.ANY),
            pl.BlockSpec((1, 1, DF), lambda t, be: (be[t], 0, 0)),
            pl.BlockSpec(memory_space=pl.ANY),
            pl.BlockSpec((1, 1, DM), lambda t, be: (be[t], 0, 0)),
        ],
        out_specs=pl.BlockSpec(
            (BM, DM), lambda t, be: (jnp.minimum(t, be[NBLK] - 1), 0)),
        scratch_shapes=[
            pltpu.VMEM((DM, DF), jnp.float32),
            pltpu.VMEM((DF, DM), jnp.float32),
            pltpu.VMEM((DM, DF), jnp.float32),
            pltpu.VMEM((DF, DM), jnp.float32),
            pltpu.SemaphoreType.DMA,
            pltpu.SemaphoreType.DMA,
        ],
    )
    return pl.pallas_call(
        _ffn_body,
        grid_spec=grid_spec,
        out_shape=jax.ShapeDtypeStruct((NROWS, DM), jnp.float32),
    )(bexp, xs, W1, b1.reshape(NE, 1, DF), W2, b2.reshape(NE, 1, DM))


HPW = TPW // 2


@functools.partial(
    pl.kernel,
    out_type=jax.ShapeDtypeStruct((NTOK, DM), jnp.float32),
    mesh=_SC_MESH,
    scratch_types=[
        [pltpu.VMEM((HPW,), jnp.int32) for _ in range(4)],
        [pltpu.VMEM((HPW, LANES), jnp.float32) for _ in range(4)],
        [pltpu.VMEM((HPW, DM), jnp.float32) for _ in range(4)],
        [pltpu.SemaphoreType.DMA for _ in range(2)],
    ],
)
def _combine(y_hbm, s0_hbm, s1_hbm, w0_hbm, w1_hbm, o_hbm,
             s_v, w_v, y_v, sems):
    wid = lax.axis_index("s") * NC + lax.axis_index("c")
    base = wid * TPW
    cps = []
    for hb in range(2):
        hbase = base + hb * HPW
        pltpu.sync_copy(s0_hbm.at[pl.ds(hbase, HPW)], s_v[2 * hb])
        pltpu.sync_copy(s1_hbm.at[pl.ds(hbase, HPW)], s_v[2 * hb + 1])
        pltpu.sync_copy(w0_hbm.at[pl.ds(hbase, HPW)], w_v[2 * hb])
        pltpu.sync_copy(w1_hbm.at[pl.ds(hbase, HPW)], w_v[2 * hb + 1])
        cps.append(pltpu.async_copy(y_hbm.at[s_v[2 * hb]],
                                    y_v[2 * hb], sems[hb]))
        cps.append(pltpu.async_copy(y_hbm.at[s_v[2 * hb + 1]],
                                    y_v[2 * hb + 1], sems[hb]))

    for hb in range(2):
        cps[2 * hb].wait()
        cps[2 * hb + 1].wait()
        y0_v, y1_v = y_v[2 * hb], y_v[2 * hb + 1]
        w0_v, w1_v = w_v[2 * hb], w_v[2 * hb + 1]

        def body(j, carry):
            wj0 = w0_v[j, :]
            wj1 = w1_v[j, :]
            for c in range(DM // LANES):
                sl = pl.ds(c * LANES, LANES)
                y0_v[j, sl] = y0_v[j, sl] * wj0 + y1_v[j, sl] * wj1
            return carry

        lax.fori_loop(0, HPW, body, 0)
        pltpu.sync_copy(y0_v, o_hbm.at[pl.ds(base + hb * HPW, HPW)])


def kernel(x, Wg, bg, W1, b1, W2, b2):
    xf = x.reshape(NTOK, DM)
    s0, s1, w0, w1, bexp = _router(xf, Wg, bg.reshape(1, NE))
    s0 = s0.reshape(NTOK)
    s1 = s1.reshape(NTOK)
    xs = _dispatch(xf, s0, s1)
    y = _ffn(bexp.reshape(4 * NBLK + 1), xs, W1, b1, W2, b2)
    out = _combine(y, s0, s1, w0, w1)
    return out.reshape(x.shape)


# speedup vs baseline: 1.0786x; 1.0089x over previous
import functools

import jax
import jax.numpy as jnp
from jax import lax
from jax.experimental import pallas as pl
from jax.experimental.pallas import tpu as pltpu
from jax.experimental.pallas import tpu_sc as plsc

NE = 8
NTOK = 2048
DM = 768
DF = 3072
BM = 128
NBLK = (2 * NTOK) // BM + NE
NROWS = NBLK * BM

NC = 2
NS = 16
NW = NC * NS
TPW = NTOK // NW
LANES = 16


def _router_body(x_ref, wg_ref, bg_ref, s0_ref, s1_ref, w0_ref, w1_ref,
                 bexp_ref):
    xf = x_ref[...]
    logits = jnp.dot(xf, wg_ref[...], preferred_element_type=jnp.float32)
    logits = logits + bg_ref[...]

    e_ids = lax.broadcasted_iota(jnp.int32, (NTOK, NE), 1)
    m1 = jnp.max(logits, axis=1, keepdims=True)
    i1 = jnp.argmax(logits, axis=1)
    oh1 = (e_ids == i1[:, None]).astype(jnp.float32)
    masked = jnp.where(oh1 > 0.0, jnp.float32(-1e30), logits)
    m2 = jnp.max(masked, axis=1, keepdims=True)
    i2 = jnp.argmax(masked, axis=1)
    oh2 = (e_ids == i2[:, None]).astype(jnp.float32)

    d = jnp.exp(m2[:, 0] - m1[:, 0])
    w1v = 1.0 / (1.0 + d)
    w2v = 1.0 - w1v

    PCH, CCH = NTOK // 16, 16
    p_i = lax.broadcasted_iota(jnp.int32, (PCH, PCH), 0)
    p_j = lax.broadcasted_iota(jnp.int32, (PCH, PCH), 1)
    ltrip = (p_j < p_i).astype(jnp.float32)

    def _ranks(oh):
        x3 = oh.reshape(PCH, CCH, NE)
        s = jnp.dot(ltrip, jnp.sum(x3, axis=1),
                    preferred_element_type=jnp.float32)
        acc = x3
        for sh in (1, 2, 4, 8):
            acc = acc + jnp.concatenate(
                [jnp.zeros((PCH, sh, NE), jnp.float32), acc[:, :-sh, :]],
                axis=1)
        t_excl = acc - x3
        return (s[:, None, :] + t_excl).reshape(NTOK, NE)

    tot0 = jnp.sum(oh1, axis=0)
    rank0 = jnp.sum(_ranks(oh1) * oh1, axis=1)
    rank1 = jnp.sum((_ranks(oh2) + tot0[None, :]) * oh2, axis=1)

    counts = tot0 + jnp.sum(oh2, axis=0)
    nblocks = jnp.ceil(counts * (1.0 / BM))
    a_i = lax.broadcasted_iota(jnp.int32, (NE, NE), 0)
    a_j = lax.broadcasted_iota(jnp.int32, (NE, NE), 1)
    ltri8 = (a_j < a_i).astype(jnp.float32)
    bstart = jnp.dot(ltri8, nblocks[:, None],
                     preferred_element_type=jnp.float32)[:, 0]
    bend = bstart + nblocks

    base0 = jnp.dot(oh1, bstart[:, None],
                    preferred_element_type=jnp.float32)[:, 0] * BM
    base1 = jnp.dot(oh2, bstart[:, None],
                    preferred_element_type=jnp.float32)[:, 0] * BM
    s0_ref[...] = (base0 + rank0).astype(jnp.int32)
    s1_ref[...] = (base1 + rank1).astype(jnp.int32)
    w0_ref[...] = jnp.broadcast_to(w1v[:, None], (NTOK, LANES))
    w1_ref[...] = jnp.broadcast_to(w2v[:, None], (NTOK, LANES))

    t_ids = lax.broadcasted_iota(jnp.int32, (NBLK, NE), 0).astype(jnp.float32)
    bexp = jnp.sum((t_ids >= bend[None, :]).astype(jnp.int32), axis=1)
    bexp = jnp.minimum(bexp, NE - 1)
    used = jnp.sum(nblocks).astype(jnp.int32)
    e_iota = lax.broadcasted_iota(jnp.int32, (NE,), 0)
    laste = jnp.max(jnp.where(counts > 0.0, e_iota, -1))
    t_blk = lax.broadcasted_iota(jnp.int32, (NBLK,), 0)
    bexp = jnp.where(t_blk < used, bexp, laste)

    present = (counts > 0.0).astype(jnp.float32)
    run_ord = jnp.dot(ltri8, present[:, None],
                      preferred_element_type=jnp.float32)[:, 0]
    par_e = run_ord - 2.0 * jnp.floor(run_ord * 0.5)
    e_f = e_iota.astype(jnp.float32)
    cand = jnp.where((present[None, :] > 0.0)
                     & (e_f[None, :] > e_f[:, None]),
                     e_f[None, :], jnp.float32(NE + 1))
    nxt_e = jnp.min(cand, axis=1)
    nxt_e = jnp.where(nxt_e > NE, -1.0, nxt_e)
    blk_oh = (bexp[:, None] ==
              lax.broadcasted_iota(jnp.int32, (NBLK, NE), 1)
              ).astype(jnp.float32)
    par_t = jnp.dot(blk_oh, par_e[:, None],
                    preferred_element_type=jnp.float32)[:, 0].astype(jnp.int32)
    nxt_t = jnp.dot(blk_oh, nxt_e[:, None],
                    preferred_element_type=jnp.float32)[:, 0].astype(jnp.int32)
    prev = jnp.concatenate([jnp.full((1,), -1, jnp.int32), bexp[:-1]])
    first_t = ((bexp != prev) & (t_blk < used)).astype(jnp.int32)
    bexp_ref[...] = jnp.concatenate(
        [bexp, used[None], first_t, par_t, nxt_t])


def _router(xf, Wg, bg2):
    return pl.pallas_call(
        _router_body,
        out_shape=(
            jax.ShapeDtypeStruct((NTOK,), jnp.int32),
            jax.ShapeDtypeStruct((NTOK,), jnp.int32),
            jax.ShapeDtypeStruct((NTOK, LANES), jnp.float32),
            jax.ShapeDtypeStruct((NTOK, LANES), jnp.float32),
            jax.ShapeDtypeStruct((4 * NBLK + 1,), jnp.int32),
        ),
    )(xf, Wg, bg2)


_SC_MESH = plsc.VectorSubcoreMesh(core_axis_name="c", subcore_axis_name="s")


@functools.partial(
    pl.kernel,
    out_type=jax.ShapeDtypeStruct((NROWS, DM), jnp.float32),
    mesh=_SC_MESH,
    scratch_types=[
        [pltpu.VMEM((TPW // 2,), jnp.int32) for _ in range(4)],
        [pltpu.VMEM((TPW // 2, DM), jnp.float32) for _ in range(2)],
        [pltpu.SemaphoreType.DMA for _ in range(2)],
    ],
)
def _dispatch(x_hbm, s0_hbm, s1_hbm, xs_hbm, s_v, rows_v, sems):
    wid = lax.axis_index("s") * NC + lax.axis_index("c")
    base = wid * TPW
    half = TPW // 2
    cps = []
    for hb in range(2):
        hbase = base + hb * half
        pltpu.sync_copy(s0_hbm.at[pl.ds(hbase, half)], s_v[2 * hb])
        pltpu.sync_copy(s1_hbm.at[pl.ds(hbase, half)], s_v[2 * hb + 1])
        pltpu.sync_copy(x_hbm.at[pl.ds(hbase, half)], rows_v[hb])
        cps.append(pltpu.async_copy(rows_v[hb], xs_hbm.at[s_v[2 * hb]],
                                    sems[hb]))
        cps.append(pltpu.async_copy(rows_v[hb], xs_hbm.at[s_v[2 * hb + 1]],
                                    sems[hb]))
    for cp in cps:
        cp.wait()


def _ffn_body(meta_ref, x_ref, w1_hbm, b1_ref, w2_hbm, b2_ref, o_ref,
              w1a, w2a, w1b, w2b, sema, semb):
    t = pl.program_id(0)
    used = meta_ref[NBLK]
    e = meta_ref[t]
    first = meta_ref[NBLK + 1 + t]
    par = meta_ref[2 * NBLK + 1 + t]
    nxt = meta_ref[3 * NBLK + 1 + t]

    @pl.when(t == 0)
    def _():
        pltpu.make_async_copy(w1_hbm.at[e], w1a, sema).start()
        pltpu.make_async_copy(w2_hbm.at[e], w2a, sema).start()

    @pl.when(first == 1)
    def _():
        @pl.when(par == 0)
        def _():
            pltpu.make_async_copy(w1_hbm.at[e], w1a, sema).wait()
            pltpu.make_async_copy(w2_hbm.at[e], w2a, sema).wait()

            @pl.when(nxt >= 0)
            def _():
                pltpu.make_async_copy(w1_hbm.at[nxt], w1b, semb).start()
                pltpu.make_async_copy(w2_hbm.at[nxt], w2b, semb).start()

        @pl.when(par == 1)
        def _():
            pltpu.make_async_copy(w1_hbm.at[e], w1b, semb).wait()
            pltpu.make_async_copy(w2_hbm.at[e], w2b, semb).wait()

            @pl.when(nxt >= 0)
            def _():
                pltpu.make_async_copy(w1_hbm.at[nxt], w1a, sema).start()
                pltpu.make_async_copy(w2_hbm.at[nxt], w2a, sema).start()

    @pl.when(t < used)
    def _():
        xb = x_ref[...]

        @pl.when(par == 0)
        def _():
            h = jnp.dot(xb, w1a[...], preferred_element_type=jnp.float32)
            h = h + b1_ref[0]
            h = 0.5 * h * (
                1.0 + lax.erf(h * jnp.float32(0.7071067811865476)))
            y = jnp.dot(h, w2a[...], preferred_element_type=jnp.float32)
            o_ref[...] = y + b2_ref[0]

        @pl.when(par == 1)
        def _():
            h = jnp.dot(xb, w1b[...], preferred_element_type=jnp.float32)
            h = h + b1_ref[0]
            h = 0.5 * h * (
                1.0 + lax.erf(h * jnp.float32(0.7071067811865476)))
            y = jnp.dot(h, w2b[...], preferred_element_type=jnp.float32)
            o_ref[...] = y + b2_ref[0]


def _ffn(bexp, xs, W1, b1, W2, b2):
    grid_spec = pltpu.PrefetchScalarGridSpec(
        num_scalar_prefetch=1,
        grid=(NBLK,),
        in_specs=[
            pl.BlockSpec((BM, DM),
                         lambda t, be: (jnp.minimum(t, be[NBLK] - 1), 0)),
            pl.BlockSpec(memory_space=pl.ANY),
            pl.BlockSpec((1, 1, DF), lambda t, be: (be[t], 0, 0)),
            pl.BlockSpec(memory_space=pl.ANY),
            pl.BlockSpec((1, 1, DM), lambda t, be: (be[t], 0, 0)),
        ],
        out_specs=pl.BlockSpec(
            (BM, DM), lambda t, be: (jnp.minimum(t, be[NBLK] - 1), 0)),
        scratch_shapes=[
            pltpu.VMEM((DM, DF), jnp.float32),
            pltpu.VMEM((DF, DM), jnp.float32),
            pltpu.VMEM((DM, DF), jnp.float32),
            pltpu.VMEM((DF, DM), jnp.float32),
            pltpu.SemaphoreType.DMA,
            pltpu.SemaphoreType.DMA,
        ],
    )
    return pl.pallas_call(
        _ffn_body,
        grid_spec=grid_spec,
        out_shape=jax.ShapeDtypeStruct((NROWS, DM), jnp.float32),
    )(bexp, xs, W1, b1.reshape(NE, 1, DF), W2, b2.reshape(NE, 1, DM))


HPW = TPW // 2


@functools.partial(
    pl.kernel,
    out_type=jax.ShapeDtypeStruct((NTOK, DM), jnp.float32),
    mesh=_SC_MESH,
    scratch_types=[
        [pltpu.VMEM((HPW,), jnp.int32) for _ in range(4)],
        [pltpu.VMEM((HPW, LANES), jnp.float32) for _ in range(4)],
        [pltpu.VMEM((HPW, DM), jnp.float32) for _ in range(4)],
        [pltpu.SemaphoreType.DMA for _ in range(2)],
    ],
)
def _combine(y_hbm, s0_hbm, s1_hbm, w0_hbm, w1_hbm, o_hbm,
             s_v, w_v, y_v, sems):
    wid = lax.axis_index("s") * NC + lax.axis_index("c")
    base = wid * TPW
    cps = []
    for hb in range(2):
        hbase = base + hb * HPW
        pltpu.sync_copy(s0_hbm.at[pl.ds(hbase, HPW)], s_v[2 * hb])
        pltpu.sync_copy(s1_hbm.at[pl.ds(hbase, HPW)], s_v[2 * hb + 1])
        pltpu.sync_copy(w0_hbm.at[pl.ds(hbase, HPW)], w_v[2 * hb])
        pltpu.sync_copy(w1_hbm.at[pl.ds(hbase, HPW)], w_v[2 * hb + 1])
        cps.append(pltpu.async_copy(y_hbm.at[s_v[2 * hb]],
                                    y_v[2 * hb], sems[hb]))
        cps.append(pltpu.async_copy(y_hbm.at[s_v[2 * hb + 1]],
                                    y_v[2 * hb + 1], sems[hb]))

    for hb in range(2):
        cps[2 * hb].wait()
        cps[2 * hb + 1].wait()
        y0_v, y1_v = y_v[2 * hb], y_v[2 * hb + 1]
        w0_v, w1_v = w_v[2 * hb], w_v[2 * hb + 1]

        def body(j, carry):
            wj0 = w0_v[j, :]
            wj1 = w1_v[j, :]
            for c in range(DM // LANES):
                sl = pl.ds(c * LANES, LANES)
                y0_v[j, sl] = y0_v[j, sl] * wj0 + y1_v[j, sl] * wj1
            return carry

        lax.fori_loop(0, HPW, body, 0)
        pltpu.sync_copy(y0_v, o_hbm.at[pl.ds(base + hb * HPW, HPW)])


def kernel(x, Wg, bg, W1, b1, W2, b2):
    xf = x.reshape(NTOK, DM)
    s0, s1, w0, w1, bexp = _router(xf, Wg, bg.reshape(1, NE))
    xs = _dispatch(xf, s0, s1)
    y = _ffn(bexp, xs, W1, b1, W2, b2)
    out = _combine(y, s0, s1, w0, w1)
    return out.reshape(x.shape)
